# baseline (speedup 1.0000x reference)
import jax
import jax.numpy as jnp
import numpy as np
from jax import lax
from jax.experimental import pallas as pl
from jax.experimental.pallas import tpu as pltpu
from jax.experimental.pallas import tpu_sc as plsc

VOCAB = 1000000
MAXLEN = 200
EMBED = 32
BATCH = 4096
SEQ = 200

NC = 2
NS = 16
NT = NC * NS
LANES = 128
SPB = 8
NBLK = MAXLEN // SPB
ETILES = EMBED // 8
TILE = 8 * LANES
OUT_ROWS = MAXLEN * ETILES * NT

_POSITIONS = np.array([0, 0] + list(range(2, 200)), dtype=np.int32)


def _body(xt_hbm, positions_hbm, token_hbm, pos_table_hbm, out_hbm,
          idx_v, rows_v, pos_idx_v, pos_v, tbuf_v, sem, wsem):
    w = lax.axis_index("s") * NC + lax.axis_index("c")
    col0 = w * LANES

    pltpu.sync_copy(positions_hbm, pos_idx_v)
    pltpu.async_copy(pos_table_hbm.at[pos_idx_v], pos_v, sem).wait()

    iota16 = lax.iota(jnp.int32, 16)

    def blk_body(blk, carry):
        s0 = blk * SPB
        idx_copies = [
            pltpu.async_copy(
                xt_hbm.at[pl.ds((s0 + i) * BATCH + col0, LANES)],
                idx_v.at[pl.ds(i * LANES, LANES)], sem)
            for i in range(SPB)
        ]
        for cp in idx_copies:
            cp.wait()
        row_copies = [
            pltpu.async_copy(token_hbm.at[idx_v.at[pl.ds(i * LANES, LANES)]],
                             rows_v.at[pl.ds(i * LANES, LANES)], sem)
            for i in range(SPB)
        ]
        for cp in row_copies:
            cp.wait()

        def s_body(i, carry2):
            s = s0 + i
            r0 = i * LANES
            svec = jnp.full((16,), s, dtype=jnp.int32)
            pa = plsc.load_gather(pos_v, [svec, iota16])
            pb = plsc.load_gather(pos_v, [svec, iota16 + 16])
            for r in range(LANES):
                plsc.addupdate(rows_v.at[r0 + r, pl.ds(0, 16)], pa)
                plsc.addupdate(rows_v.at[r0 + r, pl.ds(16, 16)], pb)
            for et in range(ETILES):
                for ei in range(8):
                    e = et * 8 + ei
                    evec = jnp.full((16,), e, dtype=jnp.int32)
                    for g in range(8):
                        bvec = iota16 + (r0 + g * 16)
                        v = plsc.load_gather(rows_v, [bvec, evec])
                        tbuf_v[et, pl.ds(ei * LANES + g * 16, 16)] = v
            t0 = s * (ETILES * NT) + w
            out_copies = [
                pltpu.async_copy(tbuf_v.at[et], out_hbm.at[t0 + et * NT], wsem)
                for et in range(ETILES)
            ]
            for cp in out_copies:
                cp.wait()
            return carry2

        lax.fori_loop(0, SPB, s_body, 0)
        return carry

    lax.fori_loop(0, NBLK, blk_body, 0)


@jax.jit
def kernel(x, token_table, pos_table):
    xt = jnp.transpose(x).reshape(MAXLEN * BATCH).astype(jnp.int32)
    positions = jnp.asarray(_POSITIONS)

    run = pl.kernel(
        _body,
        out_type=jax.ShapeDtypeStruct((OUT_ROWS, TILE), jnp.float32),
        mesh=plsc.VectorSubcoreMesh(core_axis_name="c", subcore_axis_name="s"),
        compiler_params=pltpu.CompilerParams(use_tc_tiling_on_sc=False,
                                             needs_layout_passes=False),
        scratch_types=[
            pltpu.VMEM((SPB * LANES,), jnp.int32),
            pltpu.VMEM((SPB * LANES, EMBED), jnp.float32),
            pltpu.VMEM((MAXLEN,), jnp.int32),
            pltpu.VMEM((MAXLEN, EMBED), jnp.float32),
            pltpu.VMEM((ETILES, TILE), jnp.float32),
            pltpu.SemaphoreType.DMA,
            pltpu.SemaphoreType.DMA,
        ],
    )
    z = run(xt, positions, token_table, pos_table)
    z5 = z.reshape(MAXLEN, ETILES, NT, 8, LANES)
    return z5.transpose(2, 4, 0, 1, 3).reshape(BATCH, SEQ, EMBED)


# speedup vs baseline: 1.0155x; 1.0155x over previous
import jax
import jax.numpy as jnp
import numpy as np
from jax import lax
from jax.experimental import pallas as pl
from jax.experimental.pallas import tpu as pltpu
from jax.experimental.pallas import tpu_sc as plsc

VOCAB = 1000000
MAXLEN = 200
EMBED = 32
BATCH = 4096
SEQ = 200

NC = 2
NS = 16
NT = NC * NS
LANES = 128
SPB = 8
NBLK = MAXLEN // SPB
ETILES = EMBED // 8
TILE = 8 * LANES
OUT_ROWS = MAXLEN * ETILES * NT

_POSITIONS = np.array([0, 0] + list(range(2, 200)), dtype=np.int32)


def _body(xt_hbm, positions_hbm, token_hbm, pos_table_hbm, out_hbm,
          idx0_v, idx1_v, rows0_v, rows1_v, pos_idx_v, pos_v, tbuf_v,
          g0, g1, w0, w1):
    w = lax.axis_index("s") * NC + lax.axis_index("c")
    col0 = w * LANES

    pltpu.sync_copy(positions_hbm, pos_idx_v)
    pltpu.async_copy(pos_table_hbm.at[pos_idx_v], pos_v, g0).wait()

    iota16 = lax.iota(jnp.int32, 16)

    def issue_block(b, idx_v, rows_v, sem):
        s0 = b * SPB
        idx_cps = [
            pltpu.async_copy(
                xt_hbm.at[pl.ds((s0 + i) * BATCH + col0, LANES)],
                idx_v.at[pl.ds(i * LANES, LANES)], sem)
            for i in range(SPB)
        ]
        for cp in idx_cps:
            cp.wait()
        for i in range(SPB):
            pltpu.async_copy(token_hbm.at[idx_v.at[pl.ds(i * LANES, LANES)]],
                             rows_v.at[pl.ds(i * LANES, LANES)], sem)

    def wait_block(idx_v, rows_v, sem):
        for i in range(SPB):
            pltpu.make_async_copy(
                token_hbm.at[idx_v.at[pl.ds(i * LANES, LANES)]],
                rows_v.at[pl.ds(i * LANES, LANES)], sem).wait()

    def wait_tbuf(q, wsem):
        for et in range(ETILES):
            pltpu.make_async_copy(tbuf_v.at[q, et], out_hbm.at[0],
                                  wsem).wait()

    def compute_block(b, rows_v):
        def seq_pair(ii, carry):
            for q in (0, 1):
                i = 2 * ii + q
                s = b * SPB + i
                r0 = i * LANES
                svec = jnp.full((16,), s, dtype=jnp.int32)
                pa = plsc.load_gather(pos_v, [svec, iota16])
                pb = plsc.load_gather(pos_v, [svec, iota16 + 16])
                for r in range(LANES):
                    plsc.addupdate(rows_v.at[r0 + r, pl.ds(0, 16)], pa)
                    plsc.addupdate(rows_v.at[r0 + r, pl.ds(16, 16)], pb)
                wsem = w0 if q == 0 else w1
                lax.cond(s >= 2, lambda: wait_tbuf(q, wsem), lambda: None)
                for et in range(ETILES):
                    for ei in range(8):
                        e = et * 8 + ei
                        evec = jnp.full((16,), e, dtype=jnp.int32)
                        for g in range(8):
                            bvec = iota16 + (r0 + g * 16)
                            v = plsc.load_gather(rows_v, [bvec, evec])
                            tbuf_v[q, et, pl.ds(ei * LANES + g * 16, 16)] = v
                t0 = s * (ETILES * NT) + w
                for et in range(ETILES):
                    pltpu.async_copy(tbuf_v.at[q, et],
                                     out_hbm.at[t0 + et * NT], wsem)
            return carry

        lax.fori_loop(0, SPB // 2, seq_pair, 0)

    issue_block(0, idx0_v, rows0_v, g0)

    def pair_body(bb, carry):
        b0 = 2 * bb
        b1 = 2 * bb + 1
        issue_block(b1, idx1_v, rows1_v, g1)
        wait_block(idx0_v, rows0_v, g0)
        compute_block(b0, rows0_v)
        issue_block(b1 + 1, idx0_v, rows0_v, g0)
        wait_block(idx1_v, rows1_v, g1)
        compute_block(b1, rows1_v)
        return carry

    lax.fori_loop(0, (NBLK - 1) // 2, pair_body, 0)
    wait_block(idx0_v, rows0_v, g0)
    compute_block(NBLK - 1, rows0_v)
    wait_tbuf(0, w0)
    wait_tbuf(1, w1)


@jax.jit
def kernel(x, token_table, pos_table):
    xt = jnp.transpose(x).reshape(MAXLEN * BATCH).astype(jnp.int32)
    positions = jnp.asarray(_POSITIONS)

    run = pl.kernel(
        _body,
        out_type=jax.ShapeDtypeStruct((OUT_ROWS, TILE), jnp.float32),
        mesh=plsc.VectorSubcoreMesh(core_axis_name="c", subcore_axis_name="s"),
        compiler_params=pltpu.CompilerParams(use_tc_tiling_on_sc=False,
                                             needs_layout_passes=False),
        scratch_types=[
            pltpu.VMEM((SPB * LANES,), jnp.int32),
            pltpu.VMEM((SPB * LANES,), jnp.int32),
            pltpu.VMEM((SPB * LANES, EMBED), jnp.float32),
            pltpu.VMEM((SPB * LANES, EMBED), jnp.float32),
            pltpu.VMEM((MAXLEN,), jnp.int32),
            pltpu.VMEM((MAXLEN, EMBED), jnp.float32),
            pltpu.VMEM((2, ETILES, TILE), jnp.float32),
            pltpu.SemaphoreType.DMA,
            pltpu.SemaphoreType.DMA,
            pltpu.SemaphoreType.DMA,
            pltpu.SemaphoreType.DMA,
        ],
    )
    z = run(xt, positions, token_table, pos_table)
    z5 = z.reshape(MAXLEN, ETILES, NT, 8, LANES)
    return z5.transpose(2, 4, 0, 1, 3).reshape(BATCH, SEQ, EMBED)


# speedup vs baseline: 1.0420x; 1.0261x over previous
import jax
import jax.numpy as jnp
import numpy as np
from jax import lax
from jax.experimental import pallas as pl
from jax.experimental.pallas import tpu as pltpu
from jax.experimental.pallas import tpu_sc as plsc

VOCAB = 1000000
MAXLEN = 200
EMBED = 32
BATCH = 4096
SEQ = 200

NC = 2
NS = 16
NT = NC * NS
LANES = 128
SPB = 8
NBLK = MAXLEN // SPB
ETILES = EMBED // 8
TILE = 8 * LANES
OUT_ROWS = MAXLEN * ETILES * NT
RPITCH = 40

_POSITIONS = np.array([0, 0] + list(range(2, 200)), dtype=np.int32)


def _body(xt_hbm, positions_hbm, token_hbm, pos_table_hbm, out_hbm,
          idx0_v, idx1_v, rows0_v, rows1_v, pos_idx_v, pos_v, tbuf_v, pad_v,
          g0, g1, w0, w1):
    w = lax.axis_index("s") * NC + lax.axis_index("c")
    col0 = w * LANES

    pltpu.sync_copy(positions_hbm, pos_idx_v)
    pltpu.async_copy(pos_table_hbm.at[pos_idx_v], pos_v, g0).wait()

    iota16 = lax.iota(jnp.int32, 16)

    def issue_block(b, idx_v, rows_v, sem):
        s0 = b * SPB
        idx_cps = [
            pltpu.async_copy(
                xt_hbm.at[pl.ds((s0 + i) * BATCH + col0, LANES)],
                idx_v.at[pl.ds(i * LANES, LANES)], sem)
            for i in range(SPB)
        ]
        for cp in idx_cps:
            cp.wait()
        for i in range(SPB):
            pltpu.async_copy(token_hbm.at[idx_v.at[pl.ds(i * LANES, LANES)]],
                             rows_v.at[pl.ds(i * LANES, LANES)], sem)

    def wait_block(idx_v, rows_v, sem):
        for i in range(SPB):
            pltpu.make_async_copy(
                token_hbm.at[idx_v.at[pl.ds(i * LANES, LANES)]],
                rows_v.at[pl.ds(i * LANES, LANES)], sem).wait()

    def wait_tbuf(q, wsem):
        for et in range(ETILES):
            pltpu.make_async_copy(tbuf_v.at[q, et], out_hbm.at[0],
                                  wsem).wait()

    def compute_block(b, rows_v):
        def seq_one(q):
            def body(ii, carry):
                i = 2 * ii + q
                s = b * SPB + i
                r0 = i * LANES
                svec = jnp.full((16,), s, dtype=jnp.int32)
                pa = plsc.load_gather(pos_v, [svec, iota16])
                pb = plsc.load_gather(pos_v, [svec, iota16 + 16])
                def spread(r, carry2):
                    pad_v[r, pl.ds(0, 16)] = rows_v[r0 + r, pl.ds(0, 16)] + pa
                    pad_v[r, pl.ds(16, 16)] = (
                        rows_v[r0 + r, pl.ds(16, 16)] + pb)
                    return carry2

                lax.fori_loop(0, LANES, spread, 0)
                wsem = w0 if q == 0 else w1
                lax.cond(s >= 2, lambda: wait_tbuf(q, wsem), lambda: None)
                for et in range(ETILES):
                    for ei in range(8):
                        e = et * 8 + ei
                        evec = jnp.full((16,), e, dtype=jnp.int32)
                        for g in range(8):
                            bvec = iota16 + g * 16
                            v = plsc.load_gather(pad_v, [bvec, evec])
                            tbuf_v[q, et, pl.ds(ei * LANES + g * 16, 16)] = v
                t0 = s * (ETILES * NT) + w
                for et in range(ETILES):
                    pltpu.async_copy(tbuf_v.at[q, et],
                                     out_hbm.at[t0 + et * NT], wsem)
                return carry
            return body

        lax.fori_loop(0, SPB // 2, seq_one(0), 0)
        lax.fori_loop(0, SPB // 2, seq_one(1), 0)

    issue_block(0, idx0_v, rows0_v, g0)

    def pair_body(bb, carry):
        b0 = 2 * bb
        b1 = 2 * bb + 1
        issue_block(b1, idx1_v, rows1_v, g1)
        wait_block(idx0_v, rows0_v, g0)
        compute_block(b0, rows0_v)
        issue_block(b1 + 1, idx0_v, rows0_v, g0)
        wait_block(idx1_v, rows1_v, g1)
        compute_block(b1, rows1_v)
        return carry

    lax.fori_loop(0, (NBLK - 1) // 2, pair_body, 0)
    wait_block(idx0_v, rows0_v, g0)
    compute_block(NBLK - 1, rows0_v)
    wait_tbuf(0, w0)
    wait_tbuf(1, w1)


@jax.jit
def kernel(x, token_table, pos_table):
    xt = jnp.transpose(x).reshape(MAXLEN * BATCH).astype(jnp.int32)
    positions = jnp.asarray(_POSITIONS)

    run = pl.kernel(
        _body,
        out_type=jax.ShapeDtypeStruct((OUT_ROWS, TILE), jnp.float32),
        mesh=plsc.VectorSubcoreMesh(core_axis_name="c", subcore_axis_name="s"),
        compiler_params=pltpu.CompilerParams(use_tc_tiling_on_sc=False,
                                             needs_layout_passes=False),
        scratch_types=[
            pltpu.VMEM((SPB * LANES,), jnp.int32),
            pltpu.VMEM((SPB * LANES,), jnp.int32),
            pltpu.VMEM((SPB * LANES, EMBED), jnp.float32),
            pltpu.VMEM((SPB * LANES, EMBED), jnp.float32),
            pltpu.VMEM((MAXLEN,), jnp.int32),
            pltpu.VMEM((MAXLEN, EMBED), jnp.float32),
            pltpu.VMEM((2, ETILES, TILE), jnp.float32),
            pltpu.VMEM((LANES, RPITCH), jnp.float32),
            pltpu.SemaphoreType.DMA,
            pltpu.SemaphoreType.DMA,
            pltpu.SemaphoreType.DMA,
            pltpu.SemaphoreType.DMA,
        ],
    )
    z = run(xt, positions, token_table, pos_table)
    z5 = z.reshape(MAXLEN, ETILES, NT, 8, LANES)
    return z5.transpose(2, 4, 0, 1, 3).reshape(BATCH, SEQ, EMBED)
